# baseline (speedup 1.0000x reference)
import functools

import jax
import jax.numpy as jnp
from jax import lax
from jax.experimental import pallas as pl
from jax.experimental.pallas import tpu as pltpu
from jax.experimental.pallas import tpu_sc as plsc

_B = 64
_T = 1000
_L = 16
_NC = 2
_BB = 8

_mesh = plsc.VectorSubcoreMesh(core_axis_name="c", subcore_axis_name="s", num_cores=1, num_subcores=1)


@functools.partial(
    pl.kernel,
    mesh=_mesh,
    out_type=(
        jax.ShapeDtypeStruct((_B,), jnp.float32),
        jax.ShapeDtypeStruct((_B,), jnp.float32),
    ),
    scratch_types=[
        pltpu.VMEM((_B,), jnp.int32),
        pltpu.VMEM((_T,), jnp.float32),
        pltpu.VMEM((_T,), jnp.float32),
        pltpu.VMEM((_B,), jnp.float32),
        pltpu.VMEM((_B,), jnp.float32),
        pltpu.SemaphoreType.DMA,
        pltpu.SemaphoreType.DMA,
        pltpu.SemaphoreType.DMA,
    ],
    compiler_params=pltpu.CompilerParams(needs_layout_passes=False),
)
def _sc_gather_coefs(t_hbm, a_hbm, s_hbm, a_out, s_out,
                     t_v, a_tbl, s_tbl, a_v, s_v, sem_t, sem_a, sem_s):
    wid = lax.axis_index("s") * _NC + lax.axis_index("c")
    @pl.when(wid == 0)
    def _():
        ht = pltpu.async_copy(t_hbm, t_v, sem_t)
        ha = pltpu.async_copy(a_hbm, a_tbl, sem_a)
        hs = pltpu.async_copy(s_hbm, s_tbl, sem_s)
        ht.wait()
        ha.wait()
        hs.wait()
        for j in range(_B // _L):
            idx = jax.lax.iota(jnp.int32, _L) + j * _L
            tt = plsc.load_gather(t_v, [idx])
            a_v[pl.ds(j * _L, _L)] = plsc.load_gather(a_tbl, [tt])
            s_v[pl.ds(j * _L, _L)] = plsc.load_gather(s_tbl, [tt])
        ha2 = pltpu.async_copy(a_v, a_out, sem_a)
        hs2 = pltpu.async_copy(s_v, s_out, sem_s)
        ha2.wait()
        hs2.wait()


def _qsample_body(a_ref, s_ref, x0_ref, n_ref, xt_ref):
    i = pl.program_id(0)
    for k in range(_BB):
        a = a_ref[0, i * _BB + k]
        s = s_ref[0, i * _BB + k]
        xt_ref[k] = a * x0_ref[k] + s * n_ref[k]


def kernel(x0, t, noise, sqrt_alphas_bar, sqrt_one_minus_alphas_bar):
    b, c, h, w = x0.shape
    a_coef, s_coef = _sc_gather_coefs(
        t.astype(jnp.int32), sqrt_alphas_bar, sqrt_one_minus_alphas_bar)
    xt = pl.pallas_call(
        _qsample_body,
        grid=(b // _BB,),
        in_specs=[
            pl.BlockSpec(memory_space=pltpu.SMEM),
            pl.BlockSpec(memory_space=pltpu.SMEM),
            pl.BlockSpec((_BB, c, h, w), lambda i: (i, 0, 0, 0)),
            pl.BlockSpec((_BB, c, h, w), lambda i: (i, 0, 0, 0)),
        ],
        out_specs=pl.BlockSpec((_BB, c, h, w), lambda i: (i, 0, 0, 0)),
        out_shape=jax.ShapeDtypeStruct((b, c, h, w), jnp.float32),
        compiler_params=pltpu.CompilerParams(
            dimension_semantics=("parallel",),
        ),
    )(
        a_coef.reshape(1, b),
        s_coef.reshape(1, b),
        x0,
        noise,
    )
    return xt, noise
